# baseline (speedup 1.0000x reference)
import functools

import jax
import jax.numpy as jnp
from jax.experimental import pallas as pl
import jax.experimental.pallas.tpu as pltpu


def _body(x_ref, y_ref, a_ref, b_ref, vkyw_ref, vkyb_ref, wkxw_ref,
          wkxb_ref, vkxw_ref, vkxb_ref, sig_ref, out_ref, u_ref):
    i = pl.program_id(0)

    @pl.when(i == 0)
    def _compute_u():
        cw = vkxw_ref[...] - 2.0 * wkxw_ref[...]
        cb = vkxb_ref[...] - 2.0 * wkxb_ref[...]
        u_ref[...] = (
            jnp.dot(x_ref[...], cw.T, preferred_element_type=jnp.float32)
            + cb
        )

    t = b_ref[...] + jnp.dot(
        a_ref[...], u_ref[...], preferred_element_type=jnp.float32
    )
    vky = (
        jnp.dot(y_ref[...], vkyw_ref[...].T, preferred_element_type=jnp.float32)
        + vkyb_ref[...]
    )
    out_ref[...] = jnp.maximum(vky - sig_ref[0, 0] * t, 0.0)


@functools.partial(jax.jit, static_argnames=())
def kernel(x, y, A, b, Vky_W, Vky_b, Wkx_W, Wkx_b, Vkx_W, Vkx_b, sigma):
    n, d = x.shape
    bm = 256
    grid = (n // bm,)

    full = lambda shape: pl.BlockSpec(shape, lambda i: (0, 0))
    row_blk = lambda w: pl.BlockSpec((bm, w), lambda i: (i, 0))

    out = pl.pallas_call(
        _body,
        grid=grid,
        in_specs=[
            full((n, d)),
            row_blk(d),
            row_blk(n),
            row_blk(1),
            full((d, d)),
            full((1, d)),
            full((d, d)),
            full((1, d)),
            full((d, d)),
            full((1, d)),
            pl.BlockSpec(memory_space=pltpu.SMEM),
        ],
        out_specs=row_blk(d),
        out_shape=jax.ShapeDtypeStruct((n, d), jnp.float32),
        scratch_shapes=[pltpu.VMEM((n, d), jnp.float32)],
    )(
        x, y, A, b,
        Vky_W, Vky_b.reshape(1, d),
        Wkx_W, Wkx_b.reshape(1, d),
        Vkx_W, Vkx_b.reshape(1, d),
        sigma.reshape(1, 1),
    )
    return out


# speedup vs baseline: 1.0495x; 1.0495x over previous
import functools

import jax
import jax.numpy as jnp
from jax.experimental import pallas as pl
import jax.experimental.pallas.tpu as pltpu


def _body(x_ref, y_ref, a_ref, b_ref, vkyw_ref, vkyb_ref, wkxw_ref,
          wkxb_ref, vkxw_ref, vkxb_ref, sig_ref, out_ref, u_ref):
    i = pl.program_id(0)

    @pl.when(i == 0)
    def _compute_u():
        cw = vkxw_ref[...] - 2.0 * wkxw_ref[...]
        cb = vkxb_ref[...] - 2.0 * wkxb_ref[...]
        u_ref[...] = (
            jnp.dot(x_ref[...], cw.T, preferred_element_type=jnp.float32)
            + cb
        )

    t = b_ref[...] + jnp.dot(
        a_ref[...], u_ref[...], preferred_element_type=jnp.float32
    )
    vky = (
        jnp.dot(y_ref[...], vkyw_ref[...].T, preferred_element_type=jnp.float32)
        + vkyb_ref[...]
    )
    out_ref[...] = jnp.maximum(vky - sig_ref[0, 0] * t, 0.0)


@functools.partial(jax.jit, static_argnames=())
def kernel(x, y, A, b, Vky_W, Vky_b, Wkx_W, Wkx_b, Vkx_W, Vkx_b, sigma):
    n, d = x.shape
    bm = 1024
    grid = (n // bm,)

    full = lambda shape: pl.BlockSpec(shape, lambda i: (0, 0))
    row_blk = lambda w: pl.BlockSpec((bm, w), lambda i: (i, 0))

    out = pl.pallas_call(
        _body,
        grid=grid,
        in_specs=[
            full((n, d)),
            row_blk(d),
            row_blk(n),
            row_blk(1),
            full((d, d)),
            full((1, d)),
            full((d, d)),
            full((1, d)),
            full((d, d)),
            full((1, d)),
            pl.BlockSpec(memory_space=pltpu.SMEM),
        ],
        out_specs=row_blk(d),
        out_shape=jax.ShapeDtypeStruct((n, d), jnp.float32),
        scratch_shapes=[pltpu.VMEM((n, d), jnp.float32)],
    )(
        x, y, A, b,
        Vky_W, Vky_b.reshape(1, d),
        Wkx_W, Wkx_b.reshape(1, d),
        Vkx_W, Vkx_b.reshape(1, d),
        sigma.reshape(1, 1),
    )
    return out


# speedup vs baseline: 1.1084x; 1.0561x over previous
import functools

import jax
import jax.numpy as jnp
from jax.experimental import pallas as pl
import jax.experimental.pallas.tpu as pltpu


def _body(x_ref, y_ref, a_ref, b_ref, vkyw_ref, vkyb_ref, wkxw_ref,
          wkxb_ref, vkxw_ref, vkxb_ref, sig_ref, out_ref, u_ref):
    i = pl.program_id(0)

    @pl.when(i == 0)
    def _compute_u():
        cw = vkxw_ref[...] - 2.0 * wkxw_ref[...]
        cb = vkxb_ref[...] - 2.0 * wkxb_ref[...]
        u_ref[...] = (
            jnp.dot(x_ref[...], cw.T, preferred_element_type=jnp.float32)
            + cb
        )

    t = b_ref[...] + jnp.dot(
        a_ref[...], u_ref[...], preferred_element_type=jnp.float32
    )
    vky = (
        jnp.dot(y_ref[...], vkyw_ref[...].T, preferred_element_type=jnp.float32)
        + vkyb_ref[...]
    )
    out_ref[...] = jnp.maximum(vky - sig_ref[0, 0] * t, 0.0)


@functools.partial(jax.jit, static_argnames=())
def kernel(x, y, A, b, Vky_W, Vky_b, Wkx_W, Wkx_b, Vkx_W, Vkx_b, sigma):
    n, d = x.shape
    bm = 512
    grid = (n // bm,)

    full = lambda shape: pl.BlockSpec(shape, lambda i: (0, 0))
    row_blk = lambda w: pl.BlockSpec((bm, w), lambda i: (i, 0))

    out = pl.pallas_call(
        _body,
        grid=grid,
        in_specs=[
            full((n, d)),
            row_blk(d),
            row_blk(n),
            row_blk(1),
            full((d, d)),
            full((1, d)),
            full((d, d)),
            full((1, d)),
            full((d, d)),
            full((1, d)),
            pl.BlockSpec(memory_space=pltpu.SMEM),
        ],
        out_specs=row_blk(d),
        out_shape=jax.ShapeDtypeStruct((n, d), jnp.float32),
        scratch_shapes=[pltpu.VMEM((n, d), jnp.float32)],
    )(
        x, y, A, b,
        Vky_W, Vky_b.reshape(1, d),
        Wkx_W, Wkx_b.reshape(1, d),
        Vkx_W, Vkx_b.reshape(1, d),
        sigma.reshape(1, 1),
    )
    return out
